# Initial kernel scaffold; baseline (speedup 1.0000x reference)
import jax
import jax.numpy as jnp
from jax.experimental import pallas as pl


def kernel(x, w1, g1, b1, w2, g2, b2, w3, g3, b3):
    raise NotImplementedError("write your pallas kernel here")


# speedup vs baseline: 1.1347x; 1.1347x over previous
import functools

import jax
import jax.numpy as jnp
from jax.experimental import pallas as pl
from jax.experimental.pallas import tpu as pltpu

EPS = 1e-5
_PAD = 128


def _compiler_params():
    return pltpu.CompilerParams(
        dimension_semantics=("parallel",),
        vmem_limit_bytes=64 * 1024 * 1024,
    )


def _lane_stats(y):
    return jnp.concatenate(
        [jnp.sum(y, axis=1, keepdims=True),
         jnp.sum(y * y, axis=1, keepdims=True)], axis=1)


def _fold_bn(stats, g, b, count):
    s = jnp.sum(stats, axis=0)
    mean = s[:, 0:1] / count
    var = jnp.maximum(s[:, 1:2] / count - mean * mean, 0.0)
    scale = g * jax.lax.rsqrt(var + EPS)
    shift = b - mean * scale
    return scale, shift


def _conv1_kernel(x_ref, w1_ref, t1_ref, s_ref):
    xb = x_ref[0].astype(jnp.bfloat16)
    y = jax.lax.dot_general(w1_ref[...], xb, (((0,), (0,)), ((), ())),
                            preferred_element_type=jnp.float32)
    t1_ref[0] = y.astype(jnp.bfloat16)
    s_ref[0] = _lane_stats(y)


def _conv2_kernel(w, count, t1_ref, st1_ref, g1_ref, b1_ref, w2_ref,
                  mask_ref, t2_ref, s_ref, pad_ref):
    cmid, s_len = t1_ref.shape[1], t1_ref.shape[2]
    scale, shift = _fold_bn(st1_ref[...], g1_ref[...], b1_ref[...], count)
    a = jnp.maximum(t1_ref[0].astype(jnp.float32) * scale + shift, 0.0)

    pad_ref[:, 0:_PAD] = jnp.zeros((cmid, _PAD), jnp.bfloat16)
    pad_ref[:, _PAD + s_len:] = jnp.zeros((cmid, _PAD), jnp.bfloat16)
    pad_ref[:, _PAD:_PAD + s_len] = a.astype(jnp.bfloat16)

    acc = jnp.zeros((cmid, s_len), jnp.float32)
    for k in range(9):
        dy, dx = k // 3 - 1, k % 3 - 1
        off = _PAD + dy * w + dx
        tap = pad_ref[:, off:off + s_len]
        if dx == -1:
            tap = tap * mask_ref[0:1, :]
        elif dx == 1:
            tap = tap * mask_ref[1:2, :]
        acc = acc + jax.lax.dot_general(
            w2_ref[k], tap, (((0,), (0,)), ((), ())),
            preferred_element_type=jnp.float32)
    t2_ref[0] = acc.astype(jnp.bfloat16)
    s_ref[0] = _lane_stats(acc)


def _conv3_stats_kernel(count, t2_ref, st2_ref, g2_ref, b2_ref, w3_ref,
                        s_ref):
    scale, shift = _fold_bn(st2_ref[...], g2_ref[...], b2_ref[...], count)
    a = jnp.maximum(t2_ref[0].astype(jnp.float32) * scale + shift, 0.0)
    y = jax.lax.dot_general(w3_ref[...], a.astype(jnp.bfloat16),
                            (((0,), (0,)), ((), ())),
                            preferred_element_type=jnp.float32)
    s_ref[0] = _lane_stats(y)


def _conv3_apply_kernel(count, t2_ref, st2_ref, g2_ref, b2_ref, w3_ref,
                        st3_ref, g3_ref, b3_ref, x_ref, o_ref):
    scale2, shift2 = _fold_bn(st2_ref[...], g2_ref[...], b2_ref[...], count)
    scale3, shift3 = _fold_bn(st3_ref[...], g3_ref[...], b3_ref[...], count)
    a = jnp.maximum(t2_ref[0].astype(jnp.float32) * scale2 + shift2, 0.0)
    y = jax.lax.dot_general(w3_ref[...], a.astype(jnp.bfloat16),
                            (((0,), (0,)), ((), ())),
                            preferred_element_type=jnp.float32)
    o_ref[0] = jnp.maximum(y * scale3 + shift3 + x_ref[0], 0.0)


def kernel(x, w1, g1, b1, w2, g2, b2, w3, g3, b3):
    n, cin, h, w = x.shape
    cmid = w1.shape[1]
    cout = w3.shape[1]
    s_len = h * w
    count = float(n * s_len)
    cp = _compiler_params()

    xr = x.reshape(n, cin, s_len)
    w1b = w1.astype(jnp.bfloat16)
    w2b = w2.astype(jnp.bfloat16)
    w3b = w3.astype(jnp.bfloat16)
    g1c, b1c = g1.reshape(cmid, 1), b1.reshape(cmid, 1)
    g2c, b2c = g2.reshape(cmid, 1), b2.reshape(cmid, 1)
    g3c, b3c = g3.reshape(cout, 1), b3.reshape(cout, 1)

    col = jnp.arange(s_len, dtype=jnp.int32) % w
    masks = jnp.stack([(col != 0), (col != w - 1)]).astype(jnp.bfloat16)

    rep = pl.BlockSpec((1, cmid, 2), lambda i: (i, 0, 0))
    full_stats_mid = pl.BlockSpec((n, cmid, 2), lambda i: (0, 0, 0))
    affine_mid = pl.BlockSpec((cmid, 1), lambda i: (0, 0))

    t1, s1 = pl.pallas_call(
        _conv1_kernel,
        grid=(n,),
        in_specs=[pl.BlockSpec((1, cin, s_len), lambda i: (i, 0, 0)),
                  pl.BlockSpec((cin, cmid), lambda i: (0, 0))],
        out_specs=(pl.BlockSpec((1, cmid, s_len), lambda i: (i, 0, 0)), rep),
        out_shape=(jax.ShapeDtypeStruct((n, cmid, s_len), jnp.bfloat16),
                   jax.ShapeDtypeStruct((n, cmid, 2), jnp.float32)),
        compiler_params=cp,
    )(xr, w1b)

    t2, s2 = pl.pallas_call(
        functools.partial(_conv2_kernel, w, count),
        grid=(n,),
        in_specs=[pl.BlockSpec((1, cmid, s_len), lambda i: (i, 0, 0)),
                  full_stats_mid, affine_mid, affine_mid,
                  pl.BlockSpec((9, cmid, cmid), lambda i: (0, 0, 0)),
                  pl.BlockSpec((2, s_len), lambda i: (0, 0))],
        out_specs=(pl.BlockSpec((1, cmid, s_len), lambda i: (i, 0, 0)), rep),
        out_shape=(jax.ShapeDtypeStruct((n, cmid, s_len), jnp.bfloat16),
                   jax.ShapeDtypeStruct((n, cmid, 2), jnp.float32)),
        scratch_shapes=[pltpu.VMEM((cmid, s_len + 2 * _PAD), jnp.bfloat16)],
        compiler_params=cp,
    )(t1, s1, g1c, b1c, w2b, masks)

    s3 = pl.pallas_call(
        functools.partial(_conv3_stats_kernel, count),
        grid=(n,),
        in_specs=[pl.BlockSpec((1, cmid, s_len), lambda i: (i, 0, 0)),
                  full_stats_mid, affine_mid, affine_mid,
                  pl.BlockSpec((cmid, cout), lambda i: (0, 0))],
        out_specs=pl.BlockSpec((1, cout, 2), lambda i: (i, 0, 0)),
        out_shape=jax.ShapeDtypeStruct((n, cout, 2), jnp.float32),
        compiler_params=cp,
    )(t2, s2, g2c, b2c, w3b)

    out = pl.pallas_call(
        functools.partial(_conv3_apply_kernel, count),
        grid=(n,),
        in_specs=[pl.BlockSpec((1, cmid, s_len), lambda i: (i, 0, 0)),
                  full_stats_mid, affine_mid, affine_mid,
                  pl.BlockSpec((cmid, cout), lambda i: (0, 0)),
                  pl.BlockSpec((n, cout, 2), lambda i: (0, 0, 0)),
                  pl.BlockSpec((cout, 1), lambda i: (0, 0)),
                  pl.BlockSpec((cout, 1), lambda i: (0, 0)),
                  pl.BlockSpec((1, cin, s_len), lambda i: (i, 0, 0))],
        out_specs=pl.BlockSpec((1, cout, s_len), lambda i: (i, 0, 0)),
        out_shape=jax.ShapeDtypeStruct((n, cout, s_len), jnp.float32),
        compiler_params=cp,
    )(t2, s2, g2c, b2c, w3b, s3, g3c, b3c, xr)

    return out.reshape(n, cout, h, w)


# speedup vs baseline: 1.6167x; 1.4248x over previous
import functools

import jax
import jax.numpy as jnp
from jax.experimental import pallas as pl
from jax.experimental.pallas import tpu as pltpu

EPS = 1e-5
_HALO = 8


def _compiler_params():
    return pltpu.CompilerParams(
        dimension_semantics=("parallel",),
        vmem_limit_bytes=64 * 1024 * 1024,
    )


def _row_tile(nhw, target):
    for t in range(min(target, nhw), 7, -1):
        if nhw % t == 0 and t % 8 == 0:
            return t
    return nhw


def _stats_rows(y):
    return jnp.concatenate(
        [jnp.sum(y, axis=0, keepdims=True),
         jnp.sum(y * y, axis=0, keepdims=True)], axis=0)


def _fold_bn(stats, g, b, count):
    s = jnp.sum(stats, axis=0)
    mean = s[0:1] / count
    var = jnp.maximum(s[1:2] / count - mean * mean, 0.0)
    scale = g * jax.lax.rsqrt(var + EPS)
    shift = b - mean * scale
    return scale, shift


def _conv1_kernel(x_ref, w1_ref, t1_ref, s_ref):
    y = jnp.dot(x_ref[...].astype(jnp.bfloat16), w1_ref[...],
                preferred_element_type=jnp.float32)
    t1_ref[...] = y.astype(jnp.bfloat16)
    s_ref[0] = _stats_rows(y)


def _conv2_kernel(h, w, count, t1_ref, st1_ref, g1_ref, b1_ref, w2_ref,
                  t2_ref, s_ref, pad_ref):
    cmid = t1_ref.shape[1]
    p = _HALO
    scale, shift = _fold_bn(st1_ref[...], g1_ref[...], b1_ref[...], count)
    a = jnp.maximum(t1_ref[...].astype(jnp.float32) * scale + shift, 0.0)

    pad_ref[0:1, :, :] = jnp.zeros((1, w + 2 * p, cmid), jnp.bfloat16)
    pad_ref[h + 1:h + 2, :, :] = jnp.zeros((1, w + 2 * p, cmid), jnp.bfloat16)
    pad_ref[1:h + 1, 0:p, :] = jnp.zeros((h, p, cmid), jnp.bfloat16)
    pad_ref[1:h + 1, p + w:p + w + p, :] = jnp.zeros((h, p, cmid),
                                                     jnp.bfloat16)
    pad_ref[1:h + 1, p:p + w, :] = a.astype(jnp.bfloat16).reshape(h, w, cmid)

    acc = jnp.zeros((h * w, cmid), jnp.float32)
    for k in range(9):
        oy, ox = k // 3, k % 3
        tap = pad_ref[oy:oy + h, p - 1 + ox:p - 1 + ox + w, :]
        acc = acc + jnp.dot(tap.reshape(h * w, cmid), w2_ref[k],
                            preferred_element_type=jnp.float32)
    t2_ref[...] = acc.astype(jnp.bfloat16)
    s_ref[0] = _stats_rows(acc)


def _conv3_stats_kernel(count, t2_ref, st2_ref, g2_ref, b2_ref, w3_ref,
                        s_ref):
    scale, shift = _fold_bn(st2_ref[...], g2_ref[...], b2_ref[...], count)
    a = jnp.maximum(t2_ref[...].astype(jnp.float32) * scale + shift, 0.0)
    y = jnp.dot(a.astype(jnp.bfloat16), w3_ref[...],
                preferred_element_type=jnp.float32)
    s_ref[0] = _stats_rows(y)


def _conv3_apply_kernel(count, t2_ref, st2_ref, g2_ref, b2_ref, w3_ref,
                        st3_ref, g3_ref, b3_ref, x_ref, o_ref):
    scale2, shift2 = _fold_bn(st2_ref[...], g2_ref[...], b2_ref[...], count)
    scale3, shift3 = _fold_bn(st3_ref[...], g3_ref[...], b3_ref[...], count)
    a = jnp.maximum(t2_ref[...].astype(jnp.float32) * scale2 + shift2, 0.0)
    y = jnp.dot(a.astype(jnp.bfloat16), w3_ref[...],
                preferred_element_type=jnp.float32)
    o_ref[...] = jnp.maximum(y * scale3 + shift3 + x_ref[...], 0.0)


def kernel(x, w1, g1, b1, w2, g2, b2, w3, g3, b3):
    n, cin, h, w = x.shape
    cmid = w1.shape[1]
    cout = w3.shape[1]
    nhw = n * h * w
    count = float(nhw)
    cp = _compiler_params()

    x_flat = jnp.transpose(x, (0, 2, 3, 1)).reshape(nhw, cin)
    w1b = w1.astype(jnp.bfloat16)
    w2b = w2.astype(jnp.bfloat16)
    w3b = w3.astype(jnp.bfloat16)

    tm = _row_tile(nhw, 1792)
    rt = nhw // tm
    hw = h * w

    aff_mid = pl.BlockSpec((1, cmid), lambda i: (0, 0))
    st1_full = pl.BlockSpec((rt, 2, cmid), lambda i: (0, 0, 0))
    st2_full = pl.BlockSpec((n, 2, cmid), lambda i: (0, 0, 0))

    t1, s1 = pl.pallas_call(
        _conv1_kernel,
        grid=(rt,),
        in_specs=[pl.BlockSpec((tm, cin), lambda i: (i, 0)),
                  pl.BlockSpec((cin, cmid), lambda i: (0, 0))],
        out_specs=(pl.BlockSpec((tm, cmid), lambda i: (i, 0)),
                   pl.BlockSpec((1, 2, cmid), lambda i: (i, 0, 0))),
        out_shape=(jax.ShapeDtypeStruct((nhw, cmid), jnp.bfloat16),
                   jax.ShapeDtypeStruct((rt, 2, cmid), jnp.float32)),
        compiler_params=cp,
    )(x_flat, w1b)

    t2, s2 = pl.pallas_call(
        functools.partial(_conv2_kernel, h, w, count),
        grid=(n,),
        in_specs=[pl.BlockSpec((hw, cmid), lambda i: (i, 0)),
                  st1_full, aff_mid, aff_mid,
                  pl.BlockSpec((9, cmid, cmid), lambda i: (0, 0, 0))],
        out_specs=(pl.BlockSpec((hw, cmid), lambda i: (i, 0)),
                   pl.BlockSpec((1, 2, cmid), lambda i: (i, 0, 0))),
        out_shape=(jax.ShapeDtypeStruct((nhw, cmid), jnp.bfloat16),
                   jax.ShapeDtypeStruct((n, 2, cmid), jnp.float32)),
        scratch_shapes=[pltpu.VMEM((h + 2, w + 2 * _HALO, cmid),
                                   jnp.bfloat16)],
        compiler_params=cp,
    )(t1, s1, g1, b1, w2b)

    s3 = pl.pallas_call(
        functools.partial(_conv3_stats_kernel, count),
        grid=(rt,),
        in_specs=[pl.BlockSpec((tm, cmid), lambda i: (i, 0)),
                  st2_full, aff_mid, aff_mid,
                  pl.BlockSpec((cmid, cout), lambda i: (0, 0))],
        out_specs=pl.BlockSpec((1, 2, cout), lambda i: (i, 0, 0)),
        out_shape=jax.ShapeDtypeStruct((rt, 2, cout), jnp.float32),
        compiler_params=cp,
    )(t2, s2, g2, b2, w3b)

    out = pl.pallas_call(
        functools.partial(_conv3_apply_kernel, count),
        grid=(rt,),
        in_specs=[pl.BlockSpec((tm, cmid), lambda i: (i, 0)),
                  st2_full, aff_mid, aff_mid,
                  pl.BlockSpec((cmid, cout), lambda i: (0, 0)),
                  pl.BlockSpec((rt, 2, cout), lambda i: (0, 0, 0)),
                  pl.BlockSpec((1, cout), lambda i: (0, 0)),
                  pl.BlockSpec((1, cout), lambda i: (0, 0)),
                  pl.BlockSpec((tm, cin), lambda i: (i, 0))],
        out_specs=pl.BlockSpec((tm, cout), lambda i: (i, 0)),
        out_shape=jax.ShapeDtypeStruct((nhw, cout), jnp.float32),
        compiler_params=cp,
    )(t2, s2, g2, b2, w3b, s3, g3, b3, x_flat)

    return jnp.transpose(out.reshape(n, h, w, cout), (0, 3, 1, 2))


# speedup vs baseline: 1.7354x; 1.0734x over previous
import functools

import jax
import jax.numpy as jnp
from jax.experimental import pallas as pl
from jax.experimental.pallas import tpu as pltpu

EPS = 1e-5
_HALO = 64


def _compiler_params():
    return pltpu.CompilerParams(
        dimension_semantics=("parallel",),
        vmem_limit_bytes=64 * 1024 * 1024,
    )


def _row_tile(nhw, target):
    for t in range(min(target, nhw), 7, -1):
        if nhw % t == 0 and t % 8 == 0:
            return t
    return nhw


def _stats_rows(y):
    return jnp.concatenate(
        [jnp.sum(y, axis=0, keepdims=True),
         jnp.sum(y * y, axis=0, keepdims=True)], axis=0)


def _fold_bn(stats, g, b, count):
    s = jnp.sum(stats, axis=0)
    mean = s[0:1] / count
    var = jnp.maximum(s[1:2] / count - mean * mean, 0.0)
    scale = g * jax.lax.rsqrt(var + EPS)
    shift = b - mean * scale
    return scale, shift


def _conv1_kernel(x_ref, w1_ref, t1_ref, s_ref):
    y = jnp.dot(x_ref[...].astype(jnp.bfloat16), w1_ref[...],
                preferred_element_type=jnp.float32)
    t1_ref[...] = y.astype(jnp.bfloat16)
    s_ref[0] = _stats_rows(y)


def _conv2_kernel(h, w, count, t1_ref, st1_ref, g1_ref, b1_ref, w2_ref,
                  mask_ref, t2_ref, s_ref, pad_ref):
    cmid = t1_ref.shape[1]
    hw = h * w
    p = _HALO
    scale, shift = _fold_bn(st1_ref[...], g1_ref[...], b1_ref[...], count)
    a = jnp.maximum(t1_ref[...].astype(jnp.float32) * scale + shift, 0.0)

    pad_ref[0:p, :] = jnp.zeros((p, cmid), jnp.bfloat16)
    pad_ref[p + hw:, :] = jnp.zeros((p, cmid), jnp.bfloat16)
    pad_ref[p:p + hw, :] = a.astype(jnp.bfloat16)

    acc = jnp.zeros((hw, cmid), jnp.float32)
    for k in range(9):
        dy, dx = k // 3 - 1, k % 3 - 1
        tap = pad_ref[p + dy * w + dx:p + dy * w + dx + hw, :]
        if dx == -1:
            tap = tap * mask_ref[:, 0:1]
        elif dx == 1:
            tap = tap * mask_ref[:, 1:2]
        acc = acc + jnp.dot(tap, w2_ref[k],
                            preferred_element_type=jnp.float32)
    t2_ref[...] = acc.astype(jnp.bfloat16)
    s_ref[0] = _stats_rows(acc)


def _conv3_stats_kernel(count, t2_ref, st2_ref, g2_ref, b2_ref, w3_ref,
                        s_ref):
    scale, shift = _fold_bn(st2_ref[...], g2_ref[...], b2_ref[...], count)
    a = jnp.maximum(t2_ref[...].astype(jnp.float32) * scale + shift, 0.0)
    y = jnp.dot(a.astype(jnp.bfloat16), w3_ref[...],
                preferred_element_type=jnp.float32)
    s_ref[0] = _stats_rows(y)


def _conv3_apply_kernel(count, t2_ref, st2_ref, g2_ref, b2_ref, w3_ref,
                        st3_ref, g3_ref, b3_ref, x_ref, o_ref):
    scale2, shift2 = _fold_bn(st2_ref[...], g2_ref[...], b2_ref[...], count)
    scale3, shift3 = _fold_bn(st3_ref[...], g3_ref[...], b3_ref[...], count)
    a = jnp.maximum(t2_ref[...].astype(jnp.float32) * scale2 + shift2, 0.0)
    y = jnp.dot(a.astype(jnp.bfloat16), w3_ref[...],
                preferred_element_type=jnp.float32)
    o_ref[...] = jnp.maximum(y * scale3 + shift3 + x_ref[...], 0.0)


def kernel(x, w1, g1, b1, w2, g2, b2, w3, g3, b3):
    n, cin, h, w = x.shape
    cmid = w1.shape[1]
    cout = w3.shape[1]
    nhw = n * h * w
    count = float(nhw)
    cp = _compiler_params()

    x_flat = jnp.transpose(x, (0, 2, 3, 1)).reshape(nhw, cin)
    w1b = w1.astype(jnp.bfloat16)
    w2b = w2.astype(jnp.bfloat16)
    w3b = w3.astype(jnp.bfloat16)

    tm = _row_tile(nhw, 3584)
    rt = nhw // tm
    hw = h * w

    col = jnp.arange(hw, dtype=jnp.int32) % w
    masks = jnp.stack([(col != 0), (col != w - 1)], axis=1).astype(
        jnp.bfloat16)

    aff_mid = pl.BlockSpec((1, cmid), lambda i: (0, 0))
    st1_full = pl.BlockSpec((rt, 2, cmid), lambda i: (0, 0, 0))
    st2_full = pl.BlockSpec((n, 2, cmid), lambda i: (0, 0, 0))

    t1, s1 = pl.pallas_call(
        _conv1_kernel,
        grid=(rt,),
        in_specs=[pl.BlockSpec((tm, cin), lambda i: (i, 0)),
                  pl.BlockSpec((cin, cmid), lambda i: (0, 0))],
        out_specs=(pl.BlockSpec((tm, cmid), lambda i: (i, 0)),
                   pl.BlockSpec((1, 2, cmid), lambda i: (i, 0, 0))),
        out_shape=(jax.ShapeDtypeStruct((nhw, cmid), jnp.bfloat16),
                   jax.ShapeDtypeStruct((rt, 2, cmid), jnp.float32)),
        compiler_params=cp,
    )(x_flat, w1b)

    t2, s2 = pl.pallas_call(
        functools.partial(_conv2_kernel, h, w, count),
        grid=(n,),
        in_specs=[pl.BlockSpec((hw, cmid), lambda i: (i, 0)),
                  st1_full, aff_mid, aff_mid,
                  pl.BlockSpec((9, cmid, cmid), lambda i: (0, 0, 0)),
                  pl.BlockSpec((hw, 2), lambda i: (0, 0))],
        out_specs=(pl.BlockSpec((hw, cmid), lambda i: (i, 0)),
                   pl.BlockSpec((1, 2, cmid), lambda i: (i, 0, 0))),
        out_shape=(jax.ShapeDtypeStruct((nhw, cmid), jnp.bfloat16),
                   jax.ShapeDtypeStruct((n, 2, cmid), jnp.float32)),
        scratch_shapes=[pltpu.VMEM((hw + 2 * _HALO, cmid), jnp.bfloat16)],
        compiler_params=cp,
    )(t1, s1, g1, b1, w2b, masks)

    s3 = pl.pallas_call(
        functools.partial(_conv3_stats_kernel, count),
        grid=(rt,),
        in_specs=[pl.BlockSpec((tm, cmid), lambda i: (i, 0)),
                  st2_full, aff_mid, aff_mid,
                  pl.BlockSpec((cmid, cout), lambda i: (0, 0))],
        out_specs=pl.BlockSpec((1, 2, cout), lambda i: (i, 0, 0)),
        out_shape=jax.ShapeDtypeStruct((rt, 2, cout), jnp.float32),
        compiler_params=cp,
    )(t2, s2, g2, b2, w3b)

    out = pl.pallas_call(
        functools.partial(_conv3_apply_kernel, count),
        grid=(rt,),
        in_specs=[pl.BlockSpec((tm, cmid), lambda i: (i, 0)),
                  st2_full, aff_mid, aff_mid,
                  pl.BlockSpec((cmid, cout), lambda i: (0, 0)),
                  pl.BlockSpec((rt, 2, cout), lambda i: (0, 0, 0)),
                  pl.BlockSpec((1, cout), lambda i: (0, 0)),
                  pl.BlockSpec((1, cout), lambda i: (0, 0)),
                  pl.BlockSpec((tm, cin), lambda i: (i, 0))],
        out_specs=pl.BlockSpec((tm, cout), lambda i: (i, 0)),
        out_shape=jax.ShapeDtypeStruct((nhw, cout), jnp.float32),
        compiler_params=cp,
    )(t2, s2, g2, b2, w3b, s3, g3, b3, x_flat)

    return jnp.transpose(out.reshape(n, h, w, cout), (0, 3, 1, 2))


# speedup vs baseline: 2.3316x; 1.3436x over previous
import functools

import jax
import jax.numpy as jnp
from jax.experimental import pallas as pl
from jax.experimental.pallas import tpu as pltpu

EPS = 1e-5
_HALO = 64


def _compiler_params():
    return pltpu.CompilerParams(
        dimension_semantics=("parallel",),
        vmem_limit_bytes=64 * 1024 * 1024,
    )


def _row_tile(nhw, target):
    for t in range(min(target, nhw), 7, -1):
        if nhw % t == 0 and t % 8 == 0:
            return t
    return nhw


def _stats_rows(y):
    return jnp.concatenate(
        [jnp.sum(y, axis=0, keepdims=True),
         jnp.sum(y * y, axis=0, keepdims=True)], axis=0)


def _fold_bn(stats, g, b, count, paired=False):
    s = jnp.sum(stats, axis=0)
    if paired:
        c = s.shape[1] // 2
        s = s[:, :c] + s[:, c:]
    mean = s[0:1] / count
    var = jnp.maximum(s[1:2] / count - mean * mean, 0.0)
    scale = g * jax.lax.rsqrt(var + EPS)
    shift = b - mean * scale
    return scale, shift


def _tile2(v):
    return jnp.concatenate([v, v], axis=1)


def _block_diag2(m):
    z = jnp.zeros(m.shape, m.dtype)
    return jnp.concatenate(
        [jnp.concatenate([m, z], axis=2),
         jnp.concatenate([z, m], axis=2)], axis=1)


def _conv1_kernel(hw, x_ref, w1_ref, t1_ref, s_ref):
    y = jnp.dot(x_ref[...].astype(jnp.bfloat16), w1_ref[...],
                preferred_element_type=jnp.float32)
    s_ref[0] = _stats_rows(y)
    yb = y.astype(jnp.bfloat16)
    t1_ref[0] = jnp.concatenate([yb[:hw], yb[hw:]], axis=1)


def _conv2_kernel(w, count, t1_ref, st1_ref, g1_ref, b1_ref, w2_ref,
                  mask_ref, t2_ref, s_ref, pad_ref):
    hw, lanes = t1_ref.shape[1], t1_ref.shape[2]
    p = _HALO
    scale, shift = _fold_bn(st1_ref[...], g1_ref[...], b1_ref[...], count)
    a = jnp.maximum(t1_ref[0].astype(jnp.float32) * _tile2(scale)
                    + _tile2(shift), 0.0)

    pad_ref[0:p, :] = jnp.zeros((p, lanes), jnp.bfloat16)
    pad_ref[p + hw:, :] = jnp.zeros((p, lanes), jnp.bfloat16)
    pad_ref[p:p + hw, :] = a.astype(jnp.bfloat16)

    acc = jnp.zeros((hw, lanes), jnp.float32)
    for k in range(9):
        dy, dx = k // 3 - 1, k % 3 - 1
        tap = pad_ref[p + dy * w + dx:p + dy * w + dx + hw, :]
        if dx == -1:
            tap = tap * mask_ref[:, 0:1]
        elif dx == 1:
            tap = tap * mask_ref[:, 1:2]
        acc = acc + jnp.dot(tap, w2_ref[k],
                            preferred_element_type=jnp.float32)
    t2_ref[0] = acc.astype(jnp.bfloat16)
    s_ref[0] = _stats_rows(acc)


def _conv3_stats_kernel(count, t2_ref, st2_ref, g2_ref, b2_ref, w3_ref,
                        s_ref):
    scale, shift = _fold_bn(st2_ref[...], g2_ref[...], b2_ref[...], count,
                            paired=True)
    a = jnp.maximum(t2_ref[0].astype(jnp.float32) * _tile2(scale)
                    + _tile2(shift), 0.0)
    y = jnp.dot(a.astype(jnp.bfloat16), w3_ref[...],
                preferred_element_type=jnp.float32)
    s_ref[0] = _stats_rows(y)


def _conv3_apply_kernel(cmid, count, t2_ref, st2_ref, g2_ref, b2_ref,
                        w3_ref, st3_ref, g3_ref, b3_ref, x_ref, o_ref):
    scale2, shift2 = _fold_bn(st2_ref[...], g2_ref[...], b2_ref[...], count,
                              paired=True)
    scale3, shift3 = _fold_bn(st3_ref[...], g3_ref[...], b3_ref[...], count,
                              paired=True)
    a = jnp.maximum(t2_ref[0].astype(jnp.float32) * _tile2(scale2)
                    + _tile2(shift2), 0.0)
    ab = a.astype(jnp.bfloat16)
    a2 = jnp.concatenate([ab[:, :cmid], ab[:, cmid:]], axis=0)
    y = jnp.dot(a2, w3_ref[...],
                preferred_element_type=jnp.float32)
    o_ref[...] = jnp.maximum(y * scale3 + shift3 + x_ref[...], 0.0)


def kernel(x, w1, g1, b1, w2, g2, b2, w3, g3, b3):
    n, cin, h, w = x.shape
    cmid = w1.shape[1]
    cout = w3.shape[1]
    nhw = n * h * w
    npair = n // 2
    hw = h * w
    count = float(nhw)
    cp = _compiler_params()

    x_flat = jnp.transpose(x, (0, 2, 3, 1)).reshape(nhw, cin)
    w1b = w1.astype(jnp.bfloat16)
    w2d = _block_diag2(w2.astype(jnp.bfloat16))
    w3b = w3.astype(jnp.bfloat16)
    w3d = _block_diag2(w3b[None])[0]

    col = jnp.arange(hw, dtype=jnp.int32) % w
    masks = jnp.stack([(col != 0), (col != w - 1)], axis=1).astype(
        jnp.bfloat16)

    aff_mid = pl.BlockSpec((1, cmid), lambda i: (0, 0))
    st1_full = pl.BlockSpec((npair, 2, cmid), lambda i: (0, 0, 0))
    st2_full = pl.BlockSpec((npair, 2, 2 * cmid), lambda i: (0, 0, 0))

    t1, s1 = pl.pallas_call(
        functools.partial(_conv1_kernel, hw),
        grid=(npair,),
        in_specs=[pl.BlockSpec((2 * hw, cin), lambda i: (i, 0)),
                  pl.BlockSpec((cin, cmid), lambda i: (0, 0))],
        out_specs=(pl.BlockSpec((1, hw, 2 * cmid), lambda i: (i, 0, 0)),
                   pl.BlockSpec((1, 2, cmid), lambda i: (i, 0, 0))),
        out_shape=(jax.ShapeDtypeStruct((npair, hw, 2 * cmid),
                                        jnp.bfloat16),
                   jax.ShapeDtypeStruct((npair, 2, cmid), jnp.float32)),
        compiler_params=cp,
    )(x_flat, w1b)

    t2, s2 = pl.pallas_call(
        functools.partial(_conv2_kernel, w, count),
        grid=(npair,),
        in_specs=[pl.BlockSpec((1, hw, 2 * cmid), lambda i: (i, 0, 0)),
                  st1_full, aff_mid, aff_mid,
                  pl.BlockSpec((9, 2 * cmid, 2 * cmid),
                               lambda i: (0, 0, 0)),
                  pl.BlockSpec((hw, 2), lambda i: (0, 0))],
        out_specs=(pl.BlockSpec((1, hw, 2 * cmid), lambda i: (i, 0, 0)),
                   pl.BlockSpec((1, 2, 2 * cmid), lambda i: (i, 0, 0))),
        out_shape=(jax.ShapeDtypeStruct((npair, hw, 2 * cmid),
                                        jnp.bfloat16),
                   jax.ShapeDtypeStruct((npair, 2, 2 * cmid), jnp.float32)),
        scratch_shapes=[pltpu.VMEM((hw + 2 * _HALO, 2 * cmid),
                                   jnp.bfloat16)],
        compiler_params=cp,
    )(t1, s1, g1, b1, w2d, masks)

    s3 = pl.pallas_call(
        functools.partial(_conv3_stats_kernel, count),
        grid=(npair,),
        in_specs=[pl.BlockSpec((1, hw, 2 * cmid), lambda i: (i, 0, 0)),
                  st2_full, aff_mid, aff_mid,
                  pl.BlockSpec((2 * cmid, 2 * cout), lambda i: (0, 0))],
        out_specs=pl.BlockSpec((1, 2, 2 * cout), lambda i: (i, 0, 0)),
        out_shape=jax.ShapeDtypeStruct((npair, 2, 2 * cout), jnp.float32),
        compiler_params=cp,
    )(t2, s2, g2, b2, w3d)

    out = pl.pallas_call(
        functools.partial(_conv3_apply_kernel, cmid, count),
        grid=(npair,),
        in_specs=[pl.BlockSpec((1, hw, 2 * cmid), lambda i: (i, 0, 0)),
                  st2_full, aff_mid, aff_mid,
                  pl.BlockSpec((cmid, cout), lambda i: (0, 0)),
                  pl.BlockSpec((npair, 2, 2 * cout), lambda i: (0, 0, 0)),
                  pl.BlockSpec((1, cout), lambda i: (0, 0)),
                  pl.BlockSpec((1, cout), lambda i: (0, 0)),
                  pl.BlockSpec((2 * hw, cin), lambda i: (i, 0))],
        out_specs=pl.BlockSpec((2 * hw, cout), lambda i: (i, 0)),
        out_shape=jax.ShapeDtypeStruct((nhw, cout), jnp.float32),
        compiler_params=cp,
    )(t2, s2, g2, b2, w3b, s3, g3, b3, x_flat)

    return jnp.transpose(out.reshape(n, h, w, cout), (0, 3, 1, 2))


# speedup vs baseline: 2.4837x; 1.0652x over previous
import functools

import jax
import jax.numpy as jnp
import numpy as np
from jax.experimental import pallas as pl
from jax.experimental.pallas import tpu as pltpu

EPS = 1e-5
_HALO = 64


def _compiler_params():
    return pltpu.CompilerParams(
        dimension_semantics=("parallel",),
        vmem_limit_bytes=64 * 1024 * 1024,
    )


def _row_tile(nhw, target):
    for t in range(min(target, nhw), 7, -1):
        if nhw % t == 0 and t % 8 == 0:
            return t
    return nhw


def _stats_rows(y):
    return jnp.concatenate(
        [jnp.sum(y, axis=0, keepdims=True),
         jnp.sum(y * y, axis=0, keepdims=True)], axis=0)


def _fold_bn(stats, g, b, count, paired=False):
    s = jnp.sum(stats, axis=0)
    if paired:
        c = s.shape[1] // 2
        s = s[:, :c] + s[:, c:]
    mean = s[0:1] / count
    var = jnp.maximum(s[1:2] / count - mean * mean, 0.0)
    scale = g * jax.lax.rsqrt(var + EPS)
    shift = b - mean * scale
    return scale, shift


def _tile2(v):
    return jnp.concatenate([v, v], axis=1)


def _block_diag2(m):
    z = jnp.zeros(m.shape, m.dtype)
    return jnp.concatenate(
        [jnp.concatenate([m, z], axis=2),
         jnp.concatenate([z, m], axis=2)], axis=1)


def _conv1_kernel(hw, x_ref, w1_ref, t1_ref, s_ref):
    y = jnp.dot(x_ref[...].astype(jnp.bfloat16),
                w1_ref[...].astype(jnp.bfloat16),
                preferred_element_type=jnp.float32)
    s_ref[0] = _stats_rows(y)
    yb = y.astype(jnp.bfloat16)
    t1_ref[0] = jnp.concatenate([yb[:hw], yb[hw:]], axis=1)


def _conv2_kernel(w, count, t1_ref, st1_ref, g1_ref, b1_ref, w2_ref,
                  mask_ref, t2_ref, s_ref, pad_ref):
    hw, lanes = t1_ref.shape[1], t1_ref.shape[2]
    p = _HALO
    scale, shift = _fold_bn(st1_ref[...], g1_ref[...], b1_ref[...], count)
    a = jnp.maximum(t1_ref[0].astype(jnp.float32) * _tile2(scale)
                    + _tile2(shift), 0.0)

    pad_ref[0:p, :] = jnp.zeros((p, lanes), jnp.bfloat16)
    pad_ref[p + hw:, :] = jnp.zeros((p, lanes), jnp.bfloat16)
    pad_ref[p:p + hw, :] = a.astype(jnp.bfloat16)

    w2d = _block_diag2(w2_ref[...].astype(jnp.bfloat16))
    acc = jnp.zeros((hw, lanes), jnp.float32)
    for k in range(9):
        dy, dx = k // 3 - 1, k % 3 - 1
        tap = pad_ref[p + dy * w + dx:p + dy * w + dx + hw, :]
        if dx == -1:
            tap = tap * mask_ref[:, 0:1]
        elif dx == 1:
            tap = tap * mask_ref[:, 1:2]
        acc = acc + jnp.dot(tap, w2d[k],
                            preferred_element_type=jnp.float32)
    t2_ref[0] = acc.astype(jnp.bfloat16)
    s_ref[0] = _stats_rows(acc)


def _conv3_stats_kernel(count, t2_ref, st2_ref, g2_ref, b2_ref, w3_ref,
                        s_ref):
    scale, shift = _fold_bn(st2_ref[...], g2_ref[...], b2_ref[...], count,
                            paired=True)
    a = jnp.maximum(t2_ref[0].astype(jnp.float32) * _tile2(scale)
                    + _tile2(shift), 0.0)
    w3d = _block_diag2(w3_ref[...].astype(jnp.bfloat16)[None])[0]
    y = jnp.dot(a.astype(jnp.bfloat16), w3d,
                preferred_element_type=jnp.float32)
    s_ref[0] = _stats_rows(y)


def _conv3_apply_kernel(cmid, count, t2_ref, st2_ref, g2_ref, b2_ref,
                        w3_ref, st3_ref, g3_ref, b3_ref, x_ref, o_ref):
    scale2, shift2 = _fold_bn(st2_ref[...], g2_ref[...], b2_ref[...], count,
                              paired=True)
    scale3, shift3 = _fold_bn(st3_ref[...], g3_ref[...], b3_ref[...], count,
                              paired=True)
    a = jnp.maximum(t2_ref[0].astype(jnp.float32) * _tile2(scale2)
                    + _tile2(shift2), 0.0)
    ab = a.astype(jnp.bfloat16)
    a2 = jnp.concatenate([ab[:, :cmid], ab[:, cmid:]], axis=0)
    y = jnp.dot(a2, w3_ref[...].astype(jnp.bfloat16),
                preferred_element_type=jnp.float32)
    o_ref[...] = jnp.maximum(y * scale3 + shift3 + x_ref[...], 0.0)


def kernel(x, w1, g1, b1, w2, g2, b2, w3, g3, b3):
    n, cin, h, w = x.shape
    cmid = w1.shape[1]
    cout = w3.shape[1]
    nhw = n * h * w
    npair = n // 2
    hw = h * w
    count = float(nhw)
    cp = _compiler_params()

    x_flat = jnp.transpose(x, (0, 2, 3, 1)).reshape(nhw, cin)

    col = np.arange(hw, dtype=np.int32) % w
    masks = jnp.asarray(
        np.stack([(col != 0), (col != w - 1)], axis=1).astype(np.float32),
        dtype=jnp.bfloat16)

    aff_mid = pl.BlockSpec((1, cmid), lambda i: (0, 0))
    st1_full = pl.BlockSpec((npair, 2, cmid), lambda i: (0, 0, 0))
    st2_full = pl.BlockSpec((npair, 2, 2 * cmid), lambda i: (0, 0, 0))

    t1, s1 = pl.pallas_call(
        functools.partial(_conv1_kernel, hw),
        grid=(npair,),
        in_specs=[pl.BlockSpec((2 * hw, cin), lambda i: (i, 0)),
                  pl.BlockSpec((cin, cmid), lambda i: (0, 0))],
        out_specs=(pl.BlockSpec((1, hw, 2 * cmid), lambda i: (i, 0, 0)),
                   pl.BlockSpec((1, 2, cmid), lambda i: (i, 0, 0))),
        out_shape=(jax.ShapeDtypeStruct((npair, hw, 2 * cmid),
                                        jnp.bfloat16),
                   jax.ShapeDtypeStruct((npair, 2, cmid), jnp.float32)),
        compiler_params=cp,
    )(x_flat, w1)

    t2, s2 = pl.pallas_call(
        functools.partial(_conv2_kernel, w, count),
        grid=(npair,),
        in_specs=[pl.BlockSpec((1, hw, 2 * cmid), lambda i: (i, 0, 0)),
                  st1_full, aff_mid, aff_mid,
                  pl.BlockSpec((9, cmid, cmid), lambda i: (0, 0, 0)),
                  pl.BlockSpec((hw, 2), lambda i: (0, 0))],
        out_specs=(pl.BlockSpec((1, hw, 2 * cmid), lambda i: (i, 0, 0)),
                   pl.BlockSpec((1, 2, 2 * cmid), lambda i: (i, 0, 0))),
        out_shape=(jax.ShapeDtypeStruct((npair, hw, 2 * cmid),
                                        jnp.bfloat16),
                   jax.ShapeDtypeStruct((npair, 2, 2 * cmid), jnp.float32)),
        scratch_shapes=[pltpu.VMEM((hw + 2 * _HALO, 2 * cmid),
                                   jnp.bfloat16)],
        compiler_params=cp,
    )(t1, s1, g1, b1, w2, masks)

    s3 = pl.pallas_call(
        functools.partial(_conv3_stats_kernel, count),
        grid=(npair,),
        in_specs=[pl.BlockSpec((1, hw, 2 * cmid), lambda i: (i, 0, 0)),
                  st2_full, aff_mid, aff_mid,
                  pl.BlockSpec((cmid, cout), lambda i: (0, 0))],
        out_specs=pl.BlockSpec((1, 2, 2 * cout), lambda i: (i, 0, 0)),
        out_shape=jax.ShapeDtypeStruct((npair, 2, 2 * cout), jnp.float32),
        compiler_params=cp,
    )(t2, s2, g2, b2, w3)

    out = pl.pallas_call(
        functools.partial(_conv3_apply_kernel, cmid, count),
        grid=(npair,),
        in_specs=[pl.BlockSpec((1, hw, 2 * cmid), lambda i: (i, 0, 0)),
                  st2_full, aff_mid, aff_mid,
                  pl.BlockSpec((cmid, cout), lambda i: (0, 0)),
                  pl.BlockSpec((npair, 2, 2 * cout), lambda i: (0, 0, 0)),
                  pl.BlockSpec((1, cout), lambda i: (0, 0)),
                  pl.BlockSpec((1, cout), lambda i: (0, 0)),
                  pl.BlockSpec((2 * hw, cin), lambda i: (i, 0))],
        out_specs=pl.BlockSpec((2 * hw, cout), lambda i: (i, 0)),
        out_shape=jax.ShapeDtypeStruct((nhw, cout), jnp.float32),
        compiler_params=cp,
    )(t2, s2, g2, b2, w3, s3, g3, b3, x_flat)

    return jnp.transpose(out.reshape(n, h, w, cout), (0, 3, 1, 2))
